# baseline (speedup 1.0000x reference)
import functools

import jax
import jax.numpy as jnp
from jax import lax
from jax.experimental import pallas as pl
from jax.experimental.pallas import tpu as pltpu
from jax.experimental.pallas import tpu_sc as plsc

NS_TOT = 1600000
OUT = 100000
PAD = 100352
NC, NSUB, L = 2, 16, 16
NW = NC * NSUB
ROWS = NS_TOT // 128
RPT = ROWS // NW
EXTRA = ROWS - RPT * NW
FULL_CHUNKS = 24
TAIL_HI = RPT + 1 - FULL_CHUNKS * 16
TAIL_LO = RPT - FULL_CHUNKS * 16
SLICE = PAD // NSUB
CHUNK_BYTES = 16 * 128 * 4
EPS = 1e-7


def _sc_partials(att, idx):
    mesh = plsc.VectorSubcoreMesh(core_axis_name="c", subcore_axis_name="s")

    @functools.partial(
        pl.kernel,
        out_type=jax.ShapeDtypeStruct((NC, PAD), jnp.float32),
        mesh=mesh,
        compiler_params=pltpu.CompilerParams(
            needs_layout_passes=False, use_tc_tiling_on_sc=False),
        scratch_types=[
            pltpu.VMEM((PAD,), jnp.float32),
            pltpu.VMEM((2, 16, 128), jnp.int32),
            pltpu.VMEM((2, 16, 128), jnp.float32),
            pltpu.VMEM((1, 2048), jnp.int32),
            pltpu.VMEM((1, 128), jnp.int32),
            pltpu.VMEM_SHARED((PAD,), jnp.float32),
            pltpu.SemaphoreType.DMA,
            pltpu.SemaphoreType.DMA,
        ],
    )
    def k(att_hbm, idx_hbm, out_hbm, acc, ib, vb, cidx, lobuf, shared,
          sem_in, sem_sc):
        c = lax.axis_index("c")
        s = lax.axis_index("s")
        wid = c * NSUB + s
        zeros16 = jnp.zeros((L,), jnp.float32)
        iota16 = lax.iota(jnp.int32, L)

        def zg(g, _):
            acc[pl.ds(g * L, L)] = zeros16
            return _
        lax.fori_loop(0, PAD // L, zg, None)
        off = pl.multiple_of(s * SLICE, 8)
        pltpu.sync_copy(acc.at[pl.ds(0, SLICE)], shared.at[pl.ds(off, SLICE)])
        plsc.subcore_barrier()

        base_row = wid * RPT + jnp.minimum(wid, EXTRA)
        ntail = jnp.where(wid < EXTRA, TAIL_HI, TAIL_LO)

        pltpu.sync_copy(idx_hbm.at[pl.ds(base_row, 1)], lobuf)
        lo = lobuf[0, pl.ds(0, L)][0]

        def fire_loads(row0, p):
            rsl = pl.ds(row0, 16)
            pltpu.async_copy(idx_hbm.at[rsl], ib.at[p], sem_in)
            pltpu.async_copy(att_hbm.at[rsl], vb.at[p], sem_in)

        def wait_chunk():
            pltpu.make_async_copy(
                idx_hbm.at[pl.ds(0, 16)], ib.at[0], sem_in).wait()
            pltpu.make_async_copy(
                att_hbm.at[pl.ds(0, 16)], vb.at[0], sem_in).wait()

        def process(p, nrows):
            def jbody(j, _):
                for t in range(128 // L):
                    sl = pl.ds(t * L, L)
                    plsc.addupdate_scatter(acc, [ib[p, j, sl]], vb[p, j, sl])
                return _
            lax.fori_loop(0, nrows, jbody, None)

        fire_loads(base_row, 0)

        def loop(kk, _):
            p = lax.rem(kk, 2)
            wait_chunk()

            @pl.when(kk + 1 < FULL_CHUNKS)
            def _():
                fire_loads(base_row + (kk + 1) * 16, lax.rem(kk + 1, 2))

            process(p, 16)
            return _
        lax.fori_loop(0, FULL_CHUNKS, loop, None)

        tail_row = base_row + FULL_CHUNKS * 16

        @pl.when(wid < EXTRA)
        def _():
            tsl = pl.ds(tail_row, TAIL_HI)
            pltpu.sync_copy(idx_hbm.at[tsl], ib.at[0, pl.ds(0, TAIL_HI)])
            pltpu.sync_copy(att_hbm.at[tsl], vb.at[0, pl.ds(0, TAIL_HI)])
            process(0, TAIL_HI)

        @pl.when(wid >= EXTRA)
        def _():
            tsl = pl.ds(tail_row, TAIL_LO)
            pltpu.sync_copy(idx_hbm.at[tsl], ib.at[0, pl.ds(0, TAIL_LO)])
            pltpu.sync_copy(att_hbm.at[tsl], vb.at[0, pl.ds(0, TAIL_LO)])
            process(0, TAIL_LO)

        hi = ib[0, ntail - 1, pl.ds(112, L)][15]

        sb0 = lo // 2048
        nsb = hi // 2048 + 1 - sb0

        def sb_body(sb, _):
            sb_base = (sb0 + sb) * 2048

            def gbody(g, _):
                cidx[0, pl.ds(g * L, L)] = sb_base + g * L + iota16
                return _
            lax.fori_loop(0, 2048 // L, gbody, None)

            srcoff = pl.multiple_of(sb_base, 8)
            pltpu.sync_copy(acc.at[pl.ds(srcoff, 2048)],
                            shared.at[cidx.at[0]], add=True)
            return _
        lax.fori_loop(0, nsb, sb_body, None)

        plsc.subcore_barrier()
        pltpu.sync_copy(shared.at[pl.ds(off, SLICE)],
                        out_hbm.at[c, pl.ds(off, SLICE)])

    return k(att, idx)


def _tc_merge(p):
    def body(p_ref, o_ref):
        o_ref[...] = p_ref[0] + p_ref[1]

    return pl.pallas_call(
        body,
        out_shape=jax.ShapeDtypeStruct((PAD // 128, 128), jnp.float32),
    )(p)


def kernel(weights, ray_indices, num_rays, distances):
    att = (weights / (distances + jnp.float32(EPS))).reshape(ROWS, 128)
    idx = ray_indices.reshape(ROWS, 128)
    partials = _sc_partials(att, idx)
    merged = _tc_merge(partials.reshape(NC, PAD // 128, 128))
    return merged.reshape(PAD)[:OUT][:, None]


# speedup vs baseline: 1.5085x; 1.5085x over previous
import functools

import jax
import jax.numpy as jnp
from jax import lax
from jax.experimental import pallas as pl
from jax.experimental.pallas import tpu as pltpu
from jax.experimental.pallas import tpu_sc as plsc

NS_TOT = 1600000
OUT = 100000
PAD = 100352
NC, NSUB, L = 2, 16, 16
NW = NC * NSUB
ROWS = NS_TOT // 128
RPT = ROWS // NW
EXTRA = ROWS - RPT * NW
FULL_CHUNKS = 24
TAIL_HI = RPT + 1 - FULL_CHUNKS * 16
TAIL_LO = RPT - FULL_CHUNKS * 16
SLICE = PAD // NSUB
EPS = 1e-7


def _sc_partials(att, idx):
    mesh = plsc.VectorSubcoreMesh(core_axis_name="c", subcore_axis_name="s")

    @functools.partial(
        pl.kernel,
        out_type=jax.ShapeDtypeStruct((NC, PAD), jnp.float32),
        mesh=mesh,
        compiler_params=pltpu.CompilerParams(
            needs_layout_passes=False, use_tc_tiling_on_sc=False),
        scratch_types=[
            pltpu.VMEM((2, 16, 128), jnp.int32),
            pltpu.VMEM((2, 16, 128), jnp.float32),
            pltpu.VMEM((SLICE,), jnp.float32),
            pltpu.VMEM_SHARED((PAD,), jnp.float32),
            pltpu.SemaphoreType.DMA,
            pltpu.SemaphoreType.DMA,
        ],
    )
    def k(att_hbm, idx_hbm, out_hbm, ib, vb, zb, shared, sem_in, sem_sc):
        c = lax.axis_index("c")
        s = lax.axis_index("s")
        wid = c * NSUB + s
        zeros16 = jnp.zeros((L,), jnp.float32)

        def zg(g, _):
            zb[pl.ds(g * L, L)] = zeros16
            return _
        lax.fori_loop(0, SLICE // L, zg, None)
        off = pl.multiple_of(s * SLICE, 8)
        pltpu.sync_copy(zb, shared.at[pl.ds(off, SLICE)])
        plsc.subcore_barrier()

        base_row = wid * RPT + jnp.minimum(wid, EXTRA)

        def fire_loads(row0, p):
            rsl = pl.ds(row0, 16)
            pltpu.async_copy(idx_hbm.at[rsl], ib.at[p], sem_in)
            pltpu.async_copy(att_hbm.at[rsl], vb.at[p], sem_in)

        def wait_chunk():
            pltpu.make_async_copy(
                idx_hbm.at[pl.ds(0, 16)], ib.at[0], sem_in).wait()
            pltpu.make_async_copy(
                att_hbm.at[pl.ds(0, 16)], vb.at[0], sem_in).wait()

        def scatter(p, nrows):
            cps = [
                pltpu.async_copy(
                    vb.at[p, jj], shared.at[ib.at[p, jj]], sem_sc, add=True)
                for jj in range(nrows)
            ]
            for cp in cps:
                cp.wait()

        fire_loads(base_row, 0)

        def loop(kk, _):
            p = lax.rem(kk, 2)
            wait_chunk()

            @pl.when(kk + 1 < FULL_CHUNKS)
            def _():
                fire_loads(base_row + (kk + 1) * 16, lax.rem(kk + 1, 2))

            scatter(p, 16)
            return _
        lax.fori_loop(0, FULL_CHUNKS, loop, None)

        tail_row = base_row + FULL_CHUNKS * 16

        @pl.when(wid < EXTRA)
        def _():
            tsl = pl.ds(tail_row, TAIL_HI)
            pltpu.sync_copy(idx_hbm.at[tsl], ib.at[0, pl.ds(0, TAIL_HI)])
            pltpu.sync_copy(att_hbm.at[tsl], vb.at[0, pl.ds(0, TAIL_HI)])
            scatter(0, TAIL_HI)

        @pl.when(wid >= EXTRA)
        def _():
            tsl = pl.ds(tail_row, TAIL_LO)
            pltpu.sync_copy(idx_hbm.at[tsl], ib.at[0, pl.ds(0, TAIL_LO)])
            pltpu.sync_copy(att_hbm.at[tsl], vb.at[0, pl.ds(0, TAIL_LO)])
            scatter(0, TAIL_LO)

        plsc.subcore_barrier()
        pltpu.sync_copy(shared.at[pl.ds(off, SLICE)],
                        out_hbm.at[c, pl.ds(off, SLICE)])

    return k(att, idx)


def _tc_merge(p):
    def body(p_ref, o_ref):
        o_ref[...] = p_ref[0] + p_ref[1]

    return pl.pallas_call(
        body,
        out_shape=jax.ShapeDtypeStruct((PAD // 128, 128), jnp.float32),
    )(p)


def kernel(weights, ray_indices, num_rays, distances):
    att = (weights / (distances + jnp.float32(EPS))).reshape(ROWS, 128)
    idx = ray_indices.reshape(ROWS, 128)
    partials = _sc_partials(att, idx)
    merged = _tc_merge(partials.reshape(NC, PAD // 128, 128))
    return merged.reshape(PAD)[:OUT][:, None]
